# baseline (speedup 1.0000x reference)
import functools

import jax
import jax.numpy as jnp
from jax import lax
from jax.experimental import pallas as pl
from jax.experimental.pallas import tpu as pltpu
from jax.experimental.pallas import tpu_sc as plsc

_N = 10000
_E = 320000
_D = 128
_H = 128
_C = 10
_G = 64
_NC = 2
_NS = 16
_NW = _NC * _NS
_EPW = _E // _NW
_K = 80
_NCHUNK = _EPW // _K
_RPS = 624
_TAIL = _N - _NS * _RPS


def _acc_init_drain(c, s, acc, zeros_hbm, out_hbm, body_fn, pre_fn=None):
    if pre_fn is not None:
        pre_fn()
    pltpu.sync_copy(zeros_hbm, acc.at[pl.ds(s * _RPS, _RPS)])

    @pl.when(s == 0)
    def _zero_tail():
        pltpu.sync_copy(zeros_hbm.at[pl.ds(0, _TAIL)],
                        acc.at[pl.ds(_NS * _RPS, _TAIL)])

    plsc.subcore_barrier()
    body_fn()
    plsc.subcore_barrier()
    pltpu.sync_copy(acc.at[pl.ds(s * _RPS, _RPS)],
                    out_hbm.at[c, pl.ds(s * _RPS, _RPS)])

    @pl.when(s == 0)
    def _drain_tail():
        pltpu.sync_copy(acc.at[pl.ds(_NS * _RPS, _TAIL)],
                        out_hbm.at[c, pl.ds(_NS * _RPS, _TAIL)])


def _make_edge_scatter():
    mesh = plsc.VectorSubcoreMesh(core_axis_name="c", subcore_axis_name="s")

    @functools.partial(
        pl.kernel,
        mesh=mesh,
        out_type=jax.ShapeDtypeStruct((_NC, _N, _D), jnp.float32),
        scratch_types=[
            pltpu.VMEM((_K,), jnp.int32),
            pltpu.VMEM((_K,), jnp.int32),
            pltpu.VMEM((_K,), jnp.int32),
            pltpu.VMEM((_K,), jnp.int32),
            pltpu.VMEM((_K,), jnp.int32),
            pltpu.VMEM((_K,), jnp.int32),
            pltpu.VMEM((_K,), jnp.int32),
            pltpu.VMEM((_K,), jnp.int32),
            pltpu.VMEM((_K,), jnp.int32),
            pltpu.VMEM((_K,), jnp.int32),
            pltpu.VMEM((_K,), jnp.int32),
            pltpu.VMEM((_K,), jnp.int32),
            pltpu.VMEM((_K, _D), jnp.float32),
            pltpu.VMEM((_K, _D), jnp.float32),
            pltpu.VMEM((_K, _D), jnp.float32),
            pltpu.VMEM((_K, _D), jnp.float32),
            pltpu.VMEM_SHARED((_N, _D), jnp.float32),
            pltpu.SemaphoreType.DMA,
            pltpu.SemaphoreType.DMA,
            pltpu.SemaphoreType.DMA,
            pltpu.SemaphoreType.DMA,
            pltpu.SemaphoreType.DMA,
            pltpu.SemaphoreType.DMA,
            pltpu.SemaphoreType.DMA,
            pltpu.SemaphoreType.DMA,
            pltpu.SemaphoreType.DMA,
            pltpu.SemaphoreType.DMA,
            pltpu.SemaphoreType.DMA,
            pltpu.SemaphoreType.DMA,
        ],
    )
    def edge_scatter(src_hbm, dst_hbm, tbl_hbm, zeros_hbm, out_hbm,
                     sidx0, sidx1, sidx2, sidx3,
                     didx0, didx1, didx2, didx3,
                     didx4, didx5, didx6, didx7,
                     rows0, rows1, rows2, rows3,
                     acc, semI0, semI1, semI2, semI3,
                     semG0, semG1, semG2, semG3,
                     semS0, semS1, semS2, semS3):
        c = lax.axis_index("c")
        s = lax.axis_index("s")
        base = (c * _NS + s) * _EPW
        sidx = (sidx0, sidx1, sidx2, sidx3)
        didx = (didx0, didx1, didx2, didx3, didx4, didx5, didx6, didx7)
        rows = (rows0, rows1, rows2, rows3)
        semI = (semI0, semI1, semI2, semI3)
        semG = (semG0, semG1, semG2, semG3)
        semS = (semS0, semS1, semS2, semS3)

        def issue_idx(t, r, q):
            off = base + t * _K
            pltpu.async_copy(src_hbm.at[pl.ds(off, _K)], sidx[r], semI[r])
            pltpu.async_copy(dst_hbm.at[pl.ds(off, _K)], didx[q], semI[r])

        def wait_idx(r, q):
            pltpu.make_async_copy(src_hbm.at[pl.ds(0, _K)],
                                  sidx[r], semI[r]).wait()
            pltpu.make_async_copy(dst_hbm.at[pl.ds(0, _K)],
                                  didx[q], semI[r]).wait()

        def issue_gather(r):
            pltpu.async_copy(tbl_hbm.at[sidx[r]], rows[r], semG[r])

        def wait_gather(r):
            pltpu.make_async_copy(tbl_hbm.at[sidx[r]], rows[r],
                                  semG[r]).wait()

        def issue_scatter(r, q):
            pltpu.async_copy(rows[r], acc.at[didx[q]], semS[r], add=True)

        def wait_scatter(r, q):
            pltpu.make_async_copy(rows[r], acc.at[didx[q]], semS[r]).wait()

        def chunk_step(t, r, q, has_g2, has_i3, wait_sc):
            if has_g2:
                wait_idx((r + 2) % 4, (q + 2) % 8)
            if wait_sc:
                wait_scatter((r + 2) % 4, (q + 6) % 8)
            if has_g2:
                issue_gather((r + 2) % 4)
            wait_gather(r)
            issue_scatter(r, q)
            if has_i3:
                issue_idx(t + 3, (r + 3) % 4, (q + 3) % 8)

        def pre():
            issue_idx(0, 0, 0)
            issue_idx(1, 1, 1)
            issue_idx(2, 2, 2)
            wait_idx(0, 0)
            issue_gather(0)
            wait_idx(1, 1)
            issue_gather(1)

        def body():
            chunk_step(0, 0, 0, True, True, False)
            chunk_step(1, 1, 1, True, True, False)

            def oct(i, carry):
                t = 2 + 8 * i
                for k in range(8):
                    chunk_step(t + k, (2 + k) % 4, (2 + k) % 8,
                               True, True, True)
                return carry

            lax.fori_loop(0, (_NCHUNK - 13) // 8, oct, 0)
            for t in range(_NCHUNK - 11, _NCHUNK):
                chunk_step(t, t % 4, t % 8,
                           t <= _NCHUNK - 3, t <= _NCHUNK - 4, True)
            wait_scatter((_NCHUNK - 2) % 4, (_NCHUNK - 2) % 8)
            wait_scatter((_NCHUNK - 1) % 4, (_NCHUNK - 1) % 8)

        _acc_init_drain(c, s, acc, zeros_hbm, out_hbm, body, pre_fn=pre)

    return edge_scatter


def _make_count_scatter():
    mesh = plsc.VectorSubcoreMesh(core_axis_name="c", subcore_axis_name="s")

    @functools.partial(
        pl.kernel,
        mesh=mesh,
        out_type=jax.ShapeDtypeStruct((_NC, _N, _D), jnp.float32),
        scratch_types=[
            pltpu.VMEM((_K,), jnp.int32),
            pltpu.VMEM((_K,), jnp.int32),
            pltpu.VMEM((_K,), jnp.int32),
            pltpu.VMEM((_K,), jnp.int32),
            pltpu.VMEM((_K, _D), jnp.float32),
            pltpu.VMEM_SHARED((_N, _D), jnp.float32),
            pltpu.SemaphoreType.DMA,
            pltpu.SemaphoreType.DMA,
            pltpu.SemaphoreType.DMA,
            pltpu.SemaphoreType.DMA,
            pltpu.SemaphoreType.DMA,
            pltpu.SemaphoreType.DMA,
            pltpu.SemaphoreType.DMA,
            pltpu.SemaphoreType.DMA,
        ],
    )
    def count_scatter(dst_hbm, ones_hbm, zeros_hbm, out_hbm,
                      didx0, didx1, didx2, didx3, ones_v, acc,
                      semI0, semI1, semI2, semI3,
                      semS0, semS1, semS2, semS3):
        c = lax.axis_index("c")
        s = lax.axis_index("s")
        base = (c * _NS + s) * _EPW
        didx = (didx0, didx1, didx2, didx3)
        semI = (semI0, semI1, semI2, semI3)
        semS = (semS0, semS1, semS2, semS3)

        def issue_idx(t, b):
            pltpu.async_copy(dst_hbm.at[pl.ds(base + t * _K, _K)],
                             didx[b], semI[b])

        def wait_scatter(b):
            pltpu.make_async_copy(ones_v, acc.at[didx[b]], semS[b]).wait()

        def chunk_step(t, b, has_i2, first=False):
            pltpu.make_async_copy(dst_hbm.at[pl.ds(0, _K)],
                                  didx[b], semI[b]).wait()
            if not first:
                wait_scatter((b + 2) % 4)
            pltpu.async_copy(ones_v, acc.at[didx[b]], semS[b], add=True)
            if has_i2:
                issue_idx(t + 2, (b + 2) % 4)

        def pre():
            pltpu.sync_copy(ones_hbm, ones_v)
            issue_idx(0, 0)
            issue_idx(1, 1)

        def body():
            chunk_step(0, 0, True, first=True)
            chunk_step(1, 1, True, first=True)

            def quad(i, carry):
                t = 2 + 4 * i
                chunk_step(t, 2, True)
                chunk_step(t + 1, 3, True)
                chunk_step(t + 2, 0, True)
                chunk_step(t + 3, 1, True)
                return carry

            lax.fori_loop(0, (_NCHUNK - 5) // 4, quad, 0)
            chunk_step(_NCHUNK - 3, (_NCHUNK - 3) % 4, True)
            chunk_step(_NCHUNK - 2, (_NCHUNK - 2) % 4, False)
            chunk_step(_NCHUNK - 1, (_NCHUNK - 1) % 4, False)
            wait_scatter((_NCHUNK - 2) % 4)
            wait_scatter((_NCHUNK - 1) % 4)

        _acc_init_drain(c, s, acc, zeros_hbm, out_hbm, body, pre_fn=pre)

    return count_scatter


def _dense_layer(p, c0, c1, h, Wl, bl, Wr):
    nb = 10
    bs = _N // nb

    def body(p0_ref, p1_ref, c0_ref, c1_ref, h_ref, wl_ref, bl_ref, wr_ref,
             o_ref):
        cnt = c0_ref[...] + c1_ref[...]
        mean = (p0_ref[0] + p1_ref[0]) / jnp.maximum(cnt, 1.0)
        hh = jnp.dot(mean, wl_ref[...], preferred_element_type=jnp.float32)
        hh = hh + bl_ref[...]
        hh = hh + jnp.dot(h_ref[...], wr_ref[...],
                          preferred_element_type=jnp.float32)
        o_ref[...] = jnp.maximum(hh, 0.0)

    return pl.pallas_call(
        body,
        grid=(nb,),
        in_specs=[
            pl.BlockSpec((1, bs, _D), lambda i: (0, i, 0)),
            pl.BlockSpec((1, bs, _D), lambda i: (1, i, 0)),
            pl.BlockSpec((bs, 1), lambda i: (i, 0)),
            pl.BlockSpec((bs, 1), lambda i: (i, 0)),
            pl.BlockSpec((bs, _D), lambda i: (i, 0)),
            pl.BlockSpec((_D, _H), lambda i: (0, 0)),
            pl.BlockSpec((1, _H), lambda i: (0, 0)),
            pl.BlockSpec((_D, _H), lambda i: (0, 0)),
        ],
        out_specs=pl.BlockSpec((bs, _H), lambda i: (i, 0)),
        out_shape=jax.ShapeDtypeStruct((_N, _H), jnp.float32),
    )(p, p, c0, c1, h, Wl, bl, Wr)


def _dense_pool_head(p, c0, c1, h2, Wl, bl, Wr, batch_row, Wg, bg,
                     Wlin1, blin1, Wlin2, blin2):
    def body(p0_ref, p1_ref, c0_ref, c1_ref, h2_ref, wl_ref, bl_ref, wr_ref,
             b_ref, wg_ref, bg_ref, w1_ref, b1_ref, w2_ref, b2_ref, o_ref):
        cnt = c0_ref[...] + c1_ref[...]
        mean = (p0_ref[0] + p1_ref[0]) / jnp.maximum(cnt, 1.0)
        h = jnp.dot(mean, wl_ref[...], preferred_element_type=jnp.float32)
        h = h + bl_ref[...]
        h = h + jnp.dot(h2_ref[...], wr_ref[...],
                        preferred_element_type=jnp.float32)
        h = jnp.maximum(h, 0.0)
        gate = jnp.dot(h, wg_ref[...], preferred_element_type=jnp.float32)
        gate = gate + bg_ref[0, 0]
        e = jnp.exp(gate - jnp.max(gate))
        seg = b_ref[...]
        m = (lax.broadcasted_iota(jnp.int32, (_G, _N), 0) == seg)
        mf = m.astype(jnp.float32)
        gsum = jnp.dot(mf, e, preferred_element_type=jnp.float32)
        pooled = jnp.dot(mf, e * h, preferred_element_type=jnp.float32)
        pooled = pooled / (gsum + 1e-16)
        z = jnp.dot(pooled, w1_ref[...], preferred_element_type=jnp.float32)
        z = jnp.maximum(z + b1_ref[...], 0.0)
        o = jnp.dot(z, w2_ref[...], preferred_element_type=jnp.float32)
        o = o + b2_ref[...]
        omax = jnp.max(o, axis=1, keepdims=True)
        lse = jnp.log(jnp.sum(jnp.exp(o - omax), axis=1, keepdims=True))
        o_ref[...] = o - (lse + omax)

    return pl.pallas_call(
        body,
        grid=(1,),
        in_specs=[
            pl.BlockSpec((1, _N, _D), lambda i: (0, 0, 0)),
            pl.BlockSpec((1, _N, _D), lambda i: (1, 0, 0)),
            pl.BlockSpec((_N, 1), lambda i: (0, 0)),
            pl.BlockSpec((_N, 1), lambda i: (0, 0)),
            pl.BlockSpec((_N, _D), lambda i: (0, 0)),
            pl.BlockSpec((_D, _H), lambda i: (0, 0)),
            pl.BlockSpec((1, _H), lambda i: (0, 0)),
            pl.BlockSpec((_D, _H), lambda i: (0, 0)),
            pl.BlockSpec((1, _N), lambda i: (0, 0)),
            pl.BlockSpec((_D, 1), lambda i: (0, 0)),
            pl.BlockSpec((1, 1), lambda i: (0, 0)),
            pl.BlockSpec((_H, _H), lambda i: (0, 0)),
            pl.BlockSpec((1, _H), lambda i: (0, 0)),
            pl.BlockSpec((_H, _C), lambda i: (0, 0)),
            pl.BlockSpec((1, _C), lambda i: (0, 0)),
        ],
        out_specs=pl.BlockSpec((_G, _C), lambda i: (0, 0)),
        out_shape=jax.ShapeDtypeStruct((_G, _C), jnp.float32),
    )(p, p, c0, c1, h2, Wl, bl, Wr, batch_row, Wg, bg,
      Wlin1, blin1, Wlin2, blin2)


def kernel(x, edge_index, batch, W1l, b1l, W1r, W2l, b2l, W2r, W3l, b3l,
           W3r, Wg, bg, Wlin1, blin1, Wlin2, blin2):
    src = edge_index[0]
    dst = edge_index[1]
    zeros = jnp.zeros((_RPS, _D), jnp.float32)
    ones = jnp.ones((_K, _D), jnp.float32)
    batch_row = batch.reshape(1, _N)

    edge_scatter = _make_edge_scatter()
    count_scatter = _make_count_scatter()

    cp = count_scatter(dst, ones, zeros)
    c0 = cp[0, :, :1]
    c1 = cp[1, :, :1]

    p = edge_scatter(src, dst, x, zeros)
    h1 = _dense_layer(p, c0, c1, x, W1l, b1l.reshape(1, _H), W1r)
    p = edge_scatter(src, dst, h1, zeros)
    h2 = _dense_layer(p, c0, c1, h1, W2l, b2l.reshape(1, _H), W2r)
    p = edge_scatter(src, dst, h2, zeros)
    return _dense_pool_head(
        p, c0, c1, h2, W3l, b3l.reshape(1, _H), W3r, batch_row,
        Wg, bg.reshape(1, 1), Wlin1, blin1.reshape(1, _H),
        Wlin2, blin2.reshape(1, _C))


# speedup vs baseline: 1.0028x; 1.0028x over previous
import functools

import jax
import jax.numpy as jnp
from jax import lax
from jax.experimental import pallas as pl
from jax.experimental.pallas import tpu as pltpu
from jax.experimental.pallas import tpu_sc as plsc

_N = 10000
_E = 320000
_D = 128
_H = 128
_C = 10
_G = 64
_NC = 2
_NS = 16
_NW = _NC * _NS
_EPW = _E // _NW
_K = 80
_NCHUNK = _EPW // _K
_RPS = 624
_TAIL = _N - _NS * _RPS


def _acc_init_drain(c, s, acc, zeros_hbm, out_hbm, body_fn, pre_fn=None):
    if pre_fn is not None:
        pre_fn()
    pltpu.sync_copy(zeros_hbm, acc.at[pl.ds(s * _RPS, _RPS)])

    @pl.when(s == 0)
    def _zero_tail():
        pltpu.sync_copy(zeros_hbm.at[pl.ds(0, _TAIL)],
                        acc.at[pl.ds(_NS * _RPS, _TAIL)])

    plsc.subcore_barrier()
    body_fn()
    plsc.subcore_barrier()
    pltpu.sync_copy(acc.at[pl.ds(s * _RPS, _RPS)],
                    out_hbm.at[c, pl.ds(s * _RPS, _RPS)])

    @pl.when(s == 0)
    def _drain_tail():
        pltpu.sync_copy(acc.at[pl.ds(_NS * _RPS, _TAIL)],
                        out_hbm.at[c, pl.ds(_NS * _RPS, _TAIL)])


def _make_edge_scatter():
    mesh = plsc.VectorSubcoreMesh(core_axis_name="c", subcore_axis_name="s")

    @functools.partial(
        pl.kernel,
        mesh=mesh,
        out_type=jax.ShapeDtypeStruct((_NC, _N, _D), jnp.float32),
        scratch_types=[
            pltpu.VMEM((_K,), jnp.int32),
            pltpu.VMEM((_K,), jnp.int32),
            pltpu.VMEM((_K,), jnp.int32),
            pltpu.VMEM((_K,), jnp.int32),
            pltpu.VMEM((_K,), jnp.int32),
            pltpu.VMEM((_K,), jnp.int32),
            pltpu.VMEM((_K,), jnp.int32),
            pltpu.VMEM((_K,), jnp.int32),
            pltpu.VMEM((_K, _D), jnp.float32),
            pltpu.VMEM((_K, _D), jnp.float32),
            pltpu.VMEM((_K, _D), jnp.float32),
            pltpu.VMEM((_K, _D), jnp.float32),
            pltpu.VMEM_SHARED((_N, _D), jnp.float32),
            pltpu.SemaphoreType.DMA,
            pltpu.SemaphoreType.DMA,
            pltpu.SemaphoreType.DMA,
            pltpu.SemaphoreType.DMA,
            pltpu.SemaphoreType.DMA,
            pltpu.SemaphoreType.DMA,
            pltpu.SemaphoreType.DMA,
            pltpu.SemaphoreType.DMA,
            pltpu.SemaphoreType.DMA,
        ],
    )
    def edge_scatter(src_hbm, dst_hbm, tbl_hbm, zeros_hbm, out_hbm,
                     sidx0, sidx1, sidx2, sidx3, didx0, didx1, didx2, didx3,
                     rows0, rows1, rows2, rows3,
                     acc, semI0, semI1, semI2, semI3,
                     semG0, semG1, semG2, semG3, semS):
        c = lax.axis_index("c")
        s = lax.axis_index("s")
        base = (c * _NS + s) * _EPW
        sidx = (sidx0, sidx1, sidx2, sidx3)
        didx = (didx0, didx1, didx2, didx3)
        rows = (rows0, rows1, rows2, rows3)
        semI = (semI0, semI1, semI2, semI3)
        semG = (semG0, semG1, semG2, semG3)

        def issue_idx(t, b):
            off = base + t * _K
            pltpu.async_copy(src_hbm.at[pl.ds(off, _K)], sidx[b], semI[b])
            pltpu.async_copy(dst_hbm.at[pl.ds(off, _K)], didx[b], semI[b])

        def wait_idx(b):
            pltpu.make_async_copy(src_hbm.at[pl.ds(0, _K)],
                                  sidx[b], semI[b]).wait()
            pltpu.make_async_copy(dst_hbm.at[pl.ds(0, _K)],
                                  didx[b], semI[b]).wait()

        def issue_gather(t, b):
            pltpu.async_copy(tbl_hbm.at[sidx[b]], rows[b], semG[b])

        def wait_gather(b):
            pltpu.make_async_copy(tbl_hbm.at[sidx[b]], rows[b],
                                  semG[b]).wait()

        def issue_scatter(b):
            pltpu.async_copy(rows[b], acc.at[didx[b]], semS, add=True)

        def wait_scatter(b):
            pltpu.make_async_copy(rows[b], acc.at[didx[b]], semS).wait()

        def chunk_step(t, b, has_g2, has_i3, first=False):
            if has_g2:
                wait_idx((b + 2) % 4)
                issue_gather(t + 2, (b + 2) % 4)
            wait_gather(b)
            if not first:
                wait_scatter((b - 1) % 4)
            issue_scatter(b)
            if has_i3:
                issue_idx(t + 3, (b + 3) % 4)

        def pre():
            issue_idx(0, 0)
            issue_idx(1, 1)
            issue_idx(2, 2)
            wait_idx(0)
            issue_gather(0, 0)
            wait_idx(1)
            issue_gather(1, 1)

        def body():
            chunk_step(0, 0, True, True, first=True)

            def quad(i, carry):
                t = 1 + 4 * i
                chunk_step(t, 1, True, True)
                chunk_step(t + 1, 2, True, True)
                chunk_step(t + 2, 3, True, True)
                chunk_step(t + 3, 0, True, True)
                return carry

            lax.fori_loop(0, (_NCHUNK - 5) // 4, quad, 0)
            chunk_step(_NCHUNK - 4, (_NCHUNK - 4) % 4, True, True)
            chunk_step(_NCHUNK - 3, (_NCHUNK - 3) % 4, True, False)
            chunk_step(_NCHUNK - 2, (_NCHUNK - 2) % 4, False, False)
            chunk_step(_NCHUNK - 1, (_NCHUNK - 1) % 4, False, False)
            wait_scatter((_NCHUNK - 1) % 4)

        _acc_init_drain(c, s, acc, zeros_hbm, out_hbm, body, pre_fn=pre)

    return edge_scatter


def _make_count_scatter():
    mesh = plsc.VectorSubcoreMesh(core_axis_name="c", subcore_axis_name="s")

    @functools.partial(
        pl.kernel,
        mesh=mesh,
        out_type=jax.ShapeDtypeStruct((_NC, _N, _D), jnp.float32),
        scratch_types=[
            pltpu.VMEM((_K,), jnp.int32),
            pltpu.VMEM((_K,), jnp.int32),
            pltpu.VMEM((_K,), jnp.int32),
            pltpu.VMEM((_K,), jnp.int32),
            pltpu.VMEM((_K, _D), jnp.float32),
            pltpu.VMEM_SHARED((_N, _D), jnp.float32),
            pltpu.SemaphoreType.DMA,
            pltpu.SemaphoreType.DMA,
            pltpu.SemaphoreType.DMA,
            pltpu.SemaphoreType.DMA,
            pltpu.SemaphoreType.DMA,
            pltpu.SemaphoreType.DMA,
            pltpu.SemaphoreType.DMA,
            pltpu.SemaphoreType.DMA,
        ],
    )
    def count_scatter(dst_hbm, ones_hbm, zeros_hbm, out_hbm,
                      didx0, didx1, didx2, didx3, ones_v, acc,
                      semI0, semI1, semI2, semI3,
                      semS0, semS1, semS2, semS3):
        c = lax.axis_index("c")
        s = lax.axis_index("s")
        base = (c * _NS + s) * _EPW
        didx = (didx0, didx1, didx2, didx3)
        semI = (semI0, semI1, semI2, semI3)
        semS = (semS0, semS1, semS2, semS3)

        def issue_idx(t, b):
            pltpu.async_copy(dst_hbm.at[pl.ds(base + t * _K, _K)],
                             didx[b], semI[b])

        def wait_scatter(b):
            pltpu.make_async_copy(ones_v, acc.at[didx[b]], semS[b]).wait()

        def chunk_step(t, b, has_i2, first=False):
            pltpu.make_async_copy(dst_hbm.at[pl.ds(0, _K)],
                                  didx[b], semI[b]).wait()
            if not first:
                wait_scatter((b + 2) % 4)
            pltpu.async_copy(ones_v, acc.at[didx[b]], semS[b], add=True)
            if has_i2:
                issue_idx(t + 2, (b + 2) % 4)

        def pre():
            pltpu.sync_copy(ones_hbm, ones_v)
            issue_idx(0, 0)
            issue_idx(1, 1)

        def body():
            chunk_step(0, 0, True, first=True)
            chunk_step(1, 1, True, first=True)

            def quad(i, carry):
                t = 2 + 4 * i
                chunk_step(t, 2, True)
                chunk_step(t + 1, 3, True)
                chunk_step(t + 2, 0, True)
                chunk_step(t + 3, 1, True)
                return carry

            lax.fori_loop(0, (_NCHUNK - 5) // 4, quad, 0)
            chunk_step(_NCHUNK - 3, (_NCHUNK - 3) % 4, True)
            chunk_step(_NCHUNK - 2, (_NCHUNK - 2) % 4, False)
            chunk_step(_NCHUNK - 1, (_NCHUNK - 1) % 4, False)
            wait_scatter((_NCHUNK - 2) % 4)
            wait_scatter((_NCHUNK - 1) % 4)

        _acc_init_drain(c, s, acc, zeros_hbm, out_hbm, body, pre_fn=pre)

    return count_scatter


def _dense_layer(p, c0, c1, h, Wl, bl, Wr):
    nb = 10
    bs = _N // nb

    def body(p0_ref, p1_ref, c0_ref, c1_ref, h_ref, wl_ref, bl_ref, wr_ref,
             o_ref):
        cnt = c0_ref[...] + c1_ref[...]
        mean = (p0_ref[0] + p1_ref[0]) / jnp.maximum(cnt, 1.0)
        hh = jnp.dot(mean, wl_ref[...], preferred_element_type=jnp.float32)
        hh = hh + bl_ref[...]
        hh = hh + jnp.dot(h_ref[...], wr_ref[...],
                          preferred_element_type=jnp.float32)
        o_ref[...] = jnp.maximum(hh, 0.0)

    return pl.pallas_call(
        body,
        grid=(nb,),
        in_specs=[
            pl.BlockSpec((1, bs, _D), lambda i: (0, i, 0)),
            pl.BlockSpec((1, bs, _D), lambda i: (1, i, 0)),
            pl.BlockSpec((bs, 1), lambda i: (i, 0)),
            pl.BlockSpec((bs, 1), lambda i: (i, 0)),
            pl.BlockSpec((bs, _D), lambda i: (i, 0)),
            pl.BlockSpec((_D, _H), lambda i: (0, 0)),
            pl.BlockSpec((1, _H), lambda i: (0, 0)),
            pl.BlockSpec((_D, _H), lambda i: (0, 0)),
        ],
        out_specs=pl.BlockSpec((bs, _H), lambda i: (i, 0)),
        out_shape=jax.ShapeDtypeStruct((_N, _H), jnp.float32),
    )(p, p, c0, c1, h, Wl, bl, Wr)


def _dense_pool_head(p, c0, c1, h2, Wl, bl, Wr, batch_row, Wg, bg,
                     Wlin1, blin1, Wlin2, blin2):
    def body(p0_ref, p1_ref, c0_ref, c1_ref, h2_ref, wl_ref, bl_ref, wr_ref,
             b_ref, wg_ref, bg_ref, w1_ref, b1_ref, w2_ref, b2_ref, o_ref):
        cnt = c0_ref[...] + c1_ref[...]
        mean = (p0_ref[0] + p1_ref[0]) / jnp.maximum(cnt, 1.0)
        h = jnp.dot(mean, wl_ref[...], preferred_element_type=jnp.float32)
        h = h + bl_ref[...]
        h = h + jnp.dot(h2_ref[...], wr_ref[...],
                        preferred_element_type=jnp.float32)
        h = jnp.maximum(h, 0.0)
        gate = jnp.dot(h, wg_ref[...], preferred_element_type=jnp.float32)
        gate = gate + bg_ref[0, 0]
        e = jnp.exp(gate - jnp.max(gate))
        seg = b_ref[...]
        m = (lax.broadcasted_iota(jnp.int32, (_G, _N), 0) == seg)
        mf = m.astype(jnp.float32)
        gsum = jnp.dot(mf, e, preferred_element_type=jnp.float32)
        pooled = jnp.dot(mf, e * h, preferred_element_type=jnp.float32)
        pooled = pooled / (gsum + 1e-16)
        z = jnp.dot(pooled, w1_ref[...], preferred_element_type=jnp.float32)
        z = jnp.maximum(z + b1_ref[...], 0.0)
        o = jnp.dot(z, w2_ref[...], preferred_element_type=jnp.float32)
        o = o + b2_ref[...]
        omax = jnp.max(o, axis=1, keepdims=True)
        lse = jnp.log(jnp.sum(jnp.exp(o - omax), axis=1, keepdims=True))
        o_ref[...] = o - (lse + omax)

    return pl.pallas_call(
        body,
        grid=(1,),
        in_specs=[
            pl.BlockSpec((1, _N, _D), lambda i: (0, 0, 0)),
            pl.BlockSpec((1, _N, _D), lambda i: (1, 0, 0)),
            pl.BlockSpec((_N, 1), lambda i: (0, 0)),
            pl.BlockSpec((_N, 1), lambda i: (0, 0)),
            pl.BlockSpec((_N, _D), lambda i: (0, 0)),
            pl.BlockSpec((_D, _H), lambda i: (0, 0)),
            pl.BlockSpec((1, _H), lambda i: (0, 0)),
            pl.BlockSpec((_D, _H), lambda i: (0, 0)),
            pl.BlockSpec((1, _N), lambda i: (0, 0)),
            pl.BlockSpec((_D, 1), lambda i: (0, 0)),
            pl.BlockSpec((1, 1), lambda i: (0, 0)),
            pl.BlockSpec((_H, _H), lambda i: (0, 0)),
            pl.BlockSpec((1, _H), lambda i: (0, 0)),
            pl.BlockSpec((_H, _C), lambda i: (0, 0)),
            pl.BlockSpec((1, _C), lambda i: (0, 0)),
        ],
        out_specs=pl.BlockSpec((_G, _C), lambda i: (0, 0)),
        out_shape=jax.ShapeDtypeStruct((_G, _C), jnp.float32),
    )(p, p, c0, c1, h2, Wl, bl, Wr, batch_row, Wg, bg,
      Wlin1, blin1, Wlin2, blin2)


def kernel(x, edge_index, batch, W1l, b1l, W1r, W2l, b2l, W2r, W3l, b3l,
           W3r, Wg, bg, Wlin1, blin1, Wlin2, blin2):
    src = edge_index[0]
    dst = edge_index[1]
    zeros = jnp.zeros((_RPS, _D), jnp.float32)
    ones = jnp.ones((_K, _D), jnp.float32)
    batch_row = batch.reshape(1, _N)

    edge_scatter = _make_edge_scatter()
    count_scatter = _make_count_scatter()

    cp = count_scatter(dst, ones, zeros)
    c0 = cp[0, :, :1]
    c1 = cp[1, :, :1]

    p = edge_scatter(src, dst, x, zeros)
    h1 = _dense_layer(p, c0, c1, x, W1l, b1l.reshape(1, _H), W1r)
    p = edge_scatter(src, dst, h1, zeros)
    h2 = _dense_layer(p, c0, c1, h1, W2l, b2l.reshape(1, _H), W2r)
    p = edge_scatter(src, dst, h2, zeros)
    return _dense_pool_head(
        p, c0, c1, h2, W3l, b3l.reshape(1, _H), W3r, batch_row,
        Wg, bg.reshape(1, 1), Wlin1, blin1.reshape(1, _H),
        Wlin2, blin2.reshape(1, _C))


# speedup vs baseline: 1.0122x; 1.0094x over previous
import functools

import jax
import jax.numpy as jnp
from jax import lax
from jax.experimental import pallas as pl
from jax.experimental.pallas import tpu as pltpu
from jax.experimental.pallas import tpu_sc as plsc

_N = 10000
_E = 320000
_D = 128
_H = 128
_C = 10
_G = 64
_NC = 2
_NS = 16
_NW = _NC * _NS
_EPW = _E // _NW
_K = 80
_NCHUNK = _EPW // _K
_RPS = 624
_TAIL = _N - _NS * _RPS


def _acc_init_drain(c, s, acc, zeros_hbm, out_hbm, body_fn, pre_fn=None):
    if pre_fn is not None:
        pre_fn()
    pltpu.sync_copy(zeros_hbm, acc.at[pl.ds(s * _RPS, _RPS)])

    @pl.when(s == 0)
    def _zero_tail():
        pltpu.sync_copy(zeros_hbm.at[pl.ds(0, _TAIL)],
                        acc.at[pl.ds(_NS * _RPS, _TAIL)])

    plsc.subcore_barrier()
    body_fn()
    plsc.subcore_barrier()
    pltpu.sync_copy(acc.at[pl.ds(s * _RPS, _RPS)],
                    out_hbm.at[c, pl.ds(s * _RPS, _RPS)])

    @pl.when(s == 0)
    def _drain_tail():
        pltpu.sync_copy(acc.at[pl.ds(_NS * _RPS, _TAIL)],
                        out_hbm.at[c, pl.ds(_NS * _RPS, _TAIL)])


def _make_edge_scatter():
    mesh = plsc.VectorSubcoreMesh(core_axis_name="c", subcore_axis_name="s")

    @functools.partial(
        pl.kernel,
        mesh=mesh,
        out_type=jax.ShapeDtypeStruct((_NC, _N, _D), jnp.float32),
        scratch_types=[
            pltpu.VMEM((_K,), jnp.int32),
            pltpu.VMEM((_K,), jnp.int32),
            pltpu.VMEM((_K,), jnp.int32),
            pltpu.VMEM((_K,), jnp.int32),
            pltpu.VMEM((_K,), jnp.int32),
            pltpu.VMEM((_K,), jnp.int32),
            pltpu.VMEM((_K,), jnp.int32),
            pltpu.VMEM((_K,), jnp.int32),
            pltpu.VMEM((_K, _D), jnp.float32),
            pltpu.VMEM((_K, _D), jnp.float32),
            pltpu.VMEM((_K, _D), jnp.float32),
            pltpu.VMEM((_K, _D), jnp.float32),
            pltpu.VMEM_SHARED((_N, _D), jnp.float32),
            pltpu.SemaphoreType.DMA,
            pltpu.SemaphoreType.DMA,
            pltpu.SemaphoreType.DMA,
            pltpu.SemaphoreType.DMA,
            pltpu.SemaphoreType.DMA,
            pltpu.SemaphoreType.DMA,
            pltpu.SemaphoreType.DMA,
            pltpu.SemaphoreType.DMA,
            pltpu.SemaphoreType.DMA,
        ],
    )
    def edge_scatter(src_hbm, dst_hbm, tbl_hbm, zeros_hbm, out_hbm,
                     sidx0, sidx1, sidx2, sidx3, didx0, didx1, didx2, didx3,
                     rows0, rows1, rows2, rows3,
                     acc, semI0, semI1, semI2, semI3,
                     semG0, semG1, semG2, semG3, semS):
        c = lax.axis_index("c")
        s = lax.axis_index("s")
        base = (c * _NS + s) * _EPW
        sidx = (sidx0, sidx1, sidx2, sidx3)
        didx = (didx0, didx1, didx2, didx3)
        rows = (rows0, rows1, rows2, rows3)
        semI = (semI0, semI1, semI2, semI3)
        semG = (semG0, semG1, semG2, semG3)

        def issue_idx(t, b):
            off = base + t * _K
            pltpu.async_copy(src_hbm.at[pl.ds(off, _K)], sidx[b], semI[b])
            pltpu.async_copy(dst_hbm.at[pl.ds(off, _K)], didx[b], semI[b])

        def wait_idx(b):
            pltpu.make_async_copy(src_hbm.at[pl.ds(0, _K)],
                                  sidx[b], semI[b]).wait()
            pltpu.make_async_copy(dst_hbm.at[pl.ds(0, _K)],
                                  didx[b], semI[b]).wait()

        def issue_gather(t, b):
            pltpu.async_copy(tbl_hbm.at[sidx[b]], rows[b], semG[b])

        def wait_gather(b):
            pltpu.make_async_copy(tbl_hbm.at[sidx[b]], rows[b],
                                  semG[b]).wait()

        def issue_scatter(b):
            pltpu.async_copy(rows[b], acc.at[didx[b]], semS, add=True)

        def wait_scatter(b):
            pltpu.make_async_copy(rows[b], acc.at[didx[b]], semS).wait()

        def chunk_step(t, b, has_g2, has_i3, first=False):
            if has_g2:
                wait_idx((b + 2) % 4)
                issue_gather(t + 2, (b + 2) % 4)
            wait_gather(b)
            if not first:
                wait_scatter((b - 1) % 4)
            issue_scatter(b)
            if has_i3:
                issue_idx(t + 3, (b + 3) % 4)

        def pre():
            issue_idx(0, 0)
            issue_idx(1, 1)
            issue_idx(2, 2)
            wait_idx(0)
            issue_gather(0, 0)
            wait_idx(1)
            issue_gather(1, 1)

        def body():
            chunk_step(0, 0, True, True, first=True)

            def quad(i, carry):
                t = 1 + 4 * i
                chunk_step(t, 1, True, True)
                chunk_step(t + 1, 2, True, True)
                chunk_step(t + 2, 3, True, True)
                chunk_step(t + 3, 0, True, True)
                return carry

            lax.fori_loop(0, (_NCHUNK - 5) // 4, quad, 0)
            chunk_step(_NCHUNK - 4, (_NCHUNK - 4) % 4, True, True)
            chunk_step(_NCHUNK - 3, (_NCHUNK - 3) % 4, True, False)
            chunk_step(_NCHUNK - 2, (_NCHUNK - 2) % 4, False, False)
            chunk_step(_NCHUNK - 1, (_NCHUNK - 1) % 4, False, False)
            wait_scatter((_NCHUNK - 1) % 4)

        _acc_init_drain(c, s, acc, zeros_hbm, out_hbm, body, pre_fn=pre)

    return edge_scatter


def _make_count_scatter():
    mesh = plsc.VectorSubcoreMesh(core_axis_name="c", subcore_axis_name="s")

    @functools.partial(
        pl.kernel,
        mesh=mesh,
        out_type=jax.ShapeDtypeStruct((_NC, _N, _D), jnp.float32),
        scratch_types=[
            pltpu.VMEM((_K,), jnp.int32),
            pltpu.VMEM((_K,), jnp.int32),
            pltpu.VMEM((_K,), jnp.int32),
            pltpu.VMEM((_K,), jnp.int32),
            pltpu.VMEM((_K, _D), jnp.float32),
            pltpu.VMEM_SHARED((_N, _D), jnp.float32),
            pltpu.SemaphoreType.DMA,
            pltpu.SemaphoreType.DMA,
            pltpu.SemaphoreType.DMA,
            pltpu.SemaphoreType.DMA,
            pltpu.SemaphoreType.DMA,
            pltpu.SemaphoreType.DMA,
            pltpu.SemaphoreType.DMA,
            pltpu.SemaphoreType.DMA,
        ],
    )
    def count_scatter(dst_hbm, ones_hbm, zeros_hbm, out_hbm,
                      didx0, didx1, didx2, didx3, ones_v, acc,
                      semI0, semI1, semI2, semI3,
                      semS0, semS1, semS2, semS3):
        c = lax.axis_index("c")
        s = lax.axis_index("s")
        base = (c * _NS + s) * _EPW
        didx = (didx0, didx1, didx2, didx3)
        semI = (semI0, semI1, semI2, semI3)
        semS = (semS0, semS1, semS2, semS3)

        def issue_idx(t, b):
            pltpu.async_copy(dst_hbm.at[pl.ds(base + t * _K, _K)],
                             didx[b], semI[b])

        def wait_scatter(b):
            pltpu.make_async_copy(ones_v, acc.at[didx[b]], semS[b]).wait()

        def chunk_step(t, b, has_i2, first=False):
            pltpu.make_async_copy(dst_hbm.at[pl.ds(0, _K)],
                                  didx[b], semI[b]).wait()
            if not first:
                wait_scatter((b + 2) % 4)
            pltpu.async_copy(ones_v, acc.at[didx[b]], semS[b], add=True)
            if has_i2:
                issue_idx(t + 2, (b + 2) % 4)

        def pre():
            pltpu.sync_copy(ones_hbm, ones_v)
            issue_idx(0, 0)
            issue_idx(1, 1)

        def body():
            chunk_step(0, 0, True, first=True)
            chunk_step(1, 1, True, first=True)

            def quad(i, carry):
                t = 2 + 4 * i
                chunk_step(t, 2, True)
                chunk_step(t + 1, 3, True)
                chunk_step(t + 2, 0, True)
                chunk_step(t + 3, 1, True)
                return carry

            lax.fori_loop(0, (_NCHUNK - 5) // 4, quad, 0)
            chunk_step(_NCHUNK - 3, (_NCHUNK - 3) % 4, True)
            chunk_step(_NCHUNK - 2, (_NCHUNK - 2) % 4, False)
            chunk_step(_NCHUNK - 1, (_NCHUNK - 1) % 4, False)
            wait_scatter((_NCHUNK - 2) % 4)
            wait_scatter((_NCHUNK - 1) % 4)

        _acc_init_drain(c, s, acc, zeros_hbm, out_hbm, body, pre_fn=pre)

    return count_scatter


def _dense_layer1(p, cp, h, Wl, bl, Wr):
    nb = 10
    bs = _N // nb

    def body(p0_ref, p1_ref, c0_ref, c1_ref, h_ref, wl_ref, bl_ref, wr_ref,
             o_ref, dinv_ref):
        cnt = c0_ref[0][:, :1] + c1_ref[0][:, :1]
        dinv = 1.0 / jnp.maximum(cnt, 1.0)
        dinv_ref[...] = dinv
        mean = (p0_ref[0] + p1_ref[0]) * dinv
        hh = jnp.dot(mean, wl_ref[...], preferred_element_type=jnp.float32)
        hh = hh + bl_ref[...]
        hh = hh + jnp.dot(h_ref[...], wr_ref[...],
                          preferred_element_type=jnp.float32)
        o_ref[...] = jnp.maximum(hh, 0.0)

    return pl.pallas_call(
        body,
        grid=(nb,),
        in_specs=[
            pl.BlockSpec((1, bs, _D), lambda i: (0, i, 0)),
            pl.BlockSpec((1, bs, _D), lambda i: (1, i, 0)),
            pl.BlockSpec((1, bs, _D), lambda i: (0, i, 0)),
            pl.BlockSpec((1, bs, _D), lambda i: (1, i, 0)),
            pl.BlockSpec((bs, _D), lambda i: (i, 0)),
            pl.BlockSpec((_D, _H), lambda i: (0, 0)),
            pl.BlockSpec((1, _H), lambda i: (0, 0)),
            pl.BlockSpec((_D, _H), lambda i: (0, 0)),
        ],
        out_specs=[
            pl.BlockSpec((bs, _H), lambda i: (i, 0)),
            pl.BlockSpec((bs, 1), lambda i: (i, 0)),
        ],
        out_shape=[
            jax.ShapeDtypeStruct((_N, _H), jnp.float32),
            jax.ShapeDtypeStruct((_N, 1), jnp.float32),
        ],
    )(p, p, cp, cp, h, Wl, bl, Wr)


def _dense_layer2(p, dinv, h, Wl, bl, Wr):
    nb = 10
    bs = _N // nb

    def body(p0_ref, p1_ref, dinv_ref, h_ref, wl_ref, bl_ref, wr_ref,
             o_ref):
        mean = (p0_ref[0] + p1_ref[0]) * dinv_ref[...]
        hh = jnp.dot(mean, wl_ref[...], preferred_element_type=jnp.float32)
        hh = hh + bl_ref[...]
        hh = hh + jnp.dot(h_ref[...], wr_ref[...],
                          preferred_element_type=jnp.float32)
        o_ref[...] = jnp.maximum(hh, 0.0)

    return pl.pallas_call(
        body,
        grid=(nb,),
        in_specs=[
            pl.BlockSpec((1, bs, _D), lambda i: (0, i, 0)),
            pl.BlockSpec((1, bs, _D), lambda i: (1, i, 0)),
            pl.BlockSpec((bs, 1), lambda i: (i, 0)),
            pl.BlockSpec((bs, _D), lambda i: (i, 0)),
            pl.BlockSpec((_D, _H), lambda i: (0, 0)),
            pl.BlockSpec((1, _H), lambda i: (0, 0)),
            pl.BlockSpec((_D, _H), lambda i: (0, 0)),
        ],
        out_specs=pl.BlockSpec((bs, _H), lambda i: (i, 0)),
        out_shape=jax.ShapeDtypeStruct((_N, _H), jnp.float32),
    )(p, p, dinv, h, Wl, bl, Wr)


def _dense_pool_head(p, dinv, h2, Wl, bl, Wr, batch_row, Wg, bg,
                     Wlin1, blin1, Wlin2, blin2):
    def body(p0_ref, p1_ref, dinv_ref, h2_ref, wl_ref, bl_ref, wr_ref,
             b_ref, wg_ref, bg_ref, w1_ref, b1_ref, w2_ref, b2_ref, o_ref):
        mean = (p0_ref[0] + p1_ref[0]) * dinv_ref[...]
        h = jnp.dot(mean, wl_ref[...], preferred_element_type=jnp.float32)
        h = h + bl_ref[...]
        h = h + jnp.dot(h2_ref[...], wr_ref[...],
                        preferred_element_type=jnp.float32)
        h = jnp.maximum(h, 0.0)
        gate = jnp.dot(h, wg_ref[...], preferred_element_type=jnp.float32)
        gate = gate + bg_ref[0, 0]
        e = jnp.exp(gate - jnp.max(gate))
        seg = b_ref[...]
        m = (lax.broadcasted_iota(jnp.int32, (_G, _N), 0) == seg)
        mf = m.astype(jnp.float32)
        gsum = jnp.dot(mf, e, preferred_element_type=jnp.float32)
        pooled = jnp.dot(mf, e * h, preferred_element_type=jnp.float32)
        pooled = pooled / (gsum + 1e-16)
        z = jnp.dot(pooled, w1_ref[...], preferred_element_type=jnp.float32)
        z = jnp.maximum(z + b1_ref[...], 0.0)
        o = jnp.dot(z, w2_ref[...], preferred_element_type=jnp.float32)
        o = o + b2_ref[...]
        omax = jnp.max(o, axis=1, keepdims=True)
        lse = jnp.log(jnp.sum(jnp.exp(o - omax), axis=1, keepdims=True))
        o_ref[...] = o - (lse + omax)

    return pl.pallas_call(
        body,
        grid=(1,),
        in_specs=[
            pl.BlockSpec((1, _N, _D), lambda i: (0, 0, 0)),
            pl.BlockSpec((1, _N, _D), lambda i: (1, 0, 0)),
            pl.BlockSpec((_N, 1), lambda i: (0, 0)),
            pl.BlockSpec((_N, _D), lambda i: (0, 0)),
            pl.BlockSpec((_D, _H), lambda i: (0, 0)),
            pl.BlockSpec((1, _H), lambda i: (0, 0)),
            pl.BlockSpec((_D, _H), lambda i: (0, 0)),
            pl.BlockSpec((1, _N), lambda i: (0, 0)),
            pl.BlockSpec((_D, 1), lambda i: (0, 0)),
            pl.BlockSpec((1, 1), lambda i: (0, 0)),
            pl.BlockSpec((_H, _H), lambda i: (0, 0)),
            pl.BlockSpec((1, _H), lambda i: (0, 0)),
            pl.BlockSpec((_H, _C), lambda i: (0, 0)),
            pl.BlockSpec((1, _C), lambda i: (0, 0)),
        ],
        out_specs=pl.BlockSpec((_G, _C), lambda i: (0, 0)),
        out_shape=jax.ShapeDtypeStruct((_G, _C), jnp.float32),
    )(p, p, dinv, h2, Wl, bl, Wr, batch_row, Wg, bg,
      Wlin1, blin1, Wlin2, blin2)


def kernel(x, edge_index, batch, W1l, b1l, W1r, W2l, b2l, W2r, W3l, b3l,
           W3r, Wg, bg, Wlin1, blin1, Wlin2, blin2):
    src = edge_index[0]
    dst = edge_index[1]
    zeros = jnp.zeros((_RPS, _D), jnp.float32)
    ones = jnp.ones((_K, _D), jnp.float32)
    batch_row = batch.reshape(1, _N)

    edge_scatter = _make_edge_scatter()
    count_scatter = _make_count_scatter()

    cp = count_scatter(dst, ones, zeros)

    p = edge_scatter(src, dst, x, zeros)
    h1, dinv = _dense_layer1(p, cp, x, W1l, b1l.reshape(1, _H), W1r)
    p = edge_scatter(src, dst, h1, zeros)
    h2 = _dense_layer2(p, dinv, h1, W2l, b2l.reshape(1, _H), W2r)
    p = edge_scatter(src, dst, h2, zeros)
    return _dense_pool_head(
        p, dinv, h2, W3l, b3l.reshape(1, _H), W3r, batch_row,
        Wg, bg.reshape(1, 1), Wlin1, blin1.reshape(1, _H),
        Wlin2, blin2.reshape(1, _C))
